# baseline (speedup 1.0000x reference)
import functools

import jax
import jax.numpy as jnp
from jax import lax
from jax.experimental import pallas as pl
from jax.experimental.pallas import tpu as pltpu
from jax.experimental.pallas import tpu_sc as plsc


def _make_sc_gather(B, D, n_cores, n_subcores):
    nw = n_cores * n_subcores
    bpw = B // nw
    mesh = plsc.VectorSubcoreMesh(core_axis_name="c", subcore_axis_name="s")

    @functools.partial(
        pl.kernel,
        mesh=mesh,
        out_type=jax.ShapeDtypeStruct((B, D), jnp.float32),
        scratch_types=[
            pltpu.VMEM((bpw,), jnp.int32),
            pltpu.VMEM((bpw, D), jnp.float32),
            pltpu.SemaphoreType.DMA,
        ],
    )
    def sc_gather(idx, tab, out, idx_v, rows_v, sem):
        wid = lax.axis_index("s") * n_cores + lax.axis_index("c")
        base = wid * bpw
        pltpu.sync_copy(idx.at[pl.ds(base, bpw)], idx_v)

        def fire(c, _):
            cb = c * 16
            v = idx_v[pl.ds(cb, 16)]
            for j in range(16):
                pltpu.make_async_copy(
                    tab.at[pl.ds(v[j], 1)],
                    rows_v.at[pl.ds(cb + j, 1)], sem).start()
            return ()

        lax.fori_loop(0, bpw // 16, fire, (), unroll=False)
        pltpu.make_async_copy(tab.at[pl.ds(0, bpw)], rows_v, sem).wait()
        pltpu.sync_copy(rows_v, out.at[pl.ds(base, bpw)])

    return sc_gather


def _mlp_body(ue_r, ie_r, w1_r, b1_r, w2_r, b2_r, o_r):
    H = ue_r.shape[1]
    h = lax.dot_general(ue_r[...], w1_r[:, :H],
                        (((1,), (1,)), ((), ())),
                        preferred_element_type=jnp.float32)
    h = h + lax.dot_general(ie_r[...], w1_r[:, H:],
                            (((1,), (1,)), ((), ())),
                            preferred_element_type=jnp.float32)
    h = jnp.maximum(h + b1_r[...], 0.0)
    y = jnp.sum(h * w2_r[...], axis=1, keepdims=True) + b2_r[0, 0]
    o_r[...] = 1.0 / (1.0 + jnp.exp(-y))


def _mlp(ue, ie, W1, b1, W2, b2, blk):
    B, H = ue.shape
    grid = (B // blk,)
    return pl.pallas_call(
        _mlp_body,
        grid=grid,
        in_specs=[
            pl.BlockSpec((blk, H), lambda i: (i, 0)),
            pl.BlockSpec((blk, H), lambda i: (i, 0)),
            pl.BlockSpec((H, 2 * H), lambda i: (0, 0)),
            pl.BlockSpec((1, H), lambda i: (0, 0)),
            pl.BlockSpec((1, H), lambda i: (0, 0)),
            pl.BlockSpec((1, 1), lambda i: (0, 0)),
        ],
        out_specs=pl.BlockSpec((blk, 1), lambda i: (i, 0)),
        out_shape=jax.ShapeDtypeStruct((B, 1), jnp.float32),
    )(ue, ie, W1, b1, W2, b2)


def kernel(user_id, item_id, user_table, item_table, W1, b1, W2, b2):
    B = user_id.shape[0]
    H = user_table.shape[1]
    info = plsc.get_sparse_core_info()
    gather = _make_sc_gather(B, H, info.num_cores, info.num_subcores)
    ue = gather(user_id.astype(jnp.int32), user_table)
    ie = gather(item_id.astype(jnp.int32), item_table)
    y = _mlp(ue, ie, W1, b1.reshape(1, H), W2.reshape(1, H),
             b2.reshape(1, 1), blk=2048)
    return y.reshape(B)
